# baseline (speedup 1.0000x reference)
import functools
import math

import jax
import jax.numpy as jnp
import numpy as np
from jax import lax
from jax.experimental import pallas as pl
from jax.experimental.pallas import tpu as pltpu
from jax.experimental.pallas import tpu_sc as plsc

VOCAB = 1000000
DIM = 32
MAX_LEN = 200
BATCH = 4096
SEQ = 200

NC, NS = 2, 16
NW = NC * NS
BPW = BATCH // NW
LPC = 4
CHUNKS = SEQ // LPC
DT, DR = DIM // 8, 8
SCALE = math.sqrt(DIM)


def _make_pos_table():
    para = np.arange(MAX_LEN).reshape(-1, 1) / np.power(
        10000.0, np.arange(0, DIM, 2) / DIM)
    pos = np.zeros((MAX_LEN, DIM), dtype=np.float32)
    pos[:, 0::2] = np.sin(para)
    pos[:, 1::2] = np.cos(para)
    return pos


_MESH = plsc.VectorSubcoreMesh(core_axis_name="c", subcore_axis_name="s",
                               num_cores=NC, num_subcores=NS)

STRIPE = 800
STRIPES = VOCAB // STRIPE
SPAD = 805


@functools.partial(
    pl.kernel,
    out_type=jax.ShapeDtypeStruct((VOCAB, DIM), jnp.float32),
    mesh=_MESH,
    compiler_params=pltpu.CompilerParams(use_tc_tiling_on_sc=False,
                                         needs_layout_passes=False),
    scratch_types=[
        pltpu.VMEM((DIM, SPAD), jnp.float32),
        pltpu.VMEM((STRIPE, DIM), jnp.float32),
        pltpu.SemaphoreType.DMA,
    ],
)
def _sc_transpose(wt_hbm, wrm_hbm, src_v, dst_v, ssem):
    wid = lax.axis_index("s") * NC + lax.axis_index("c")
    iot = lax.iota(jnp.int32, 16)

    def stripe_body(k, _):
        s = k * NW + wid

        @pl.when(s < STRIPES)
        def _():
            i0 = pl.multiple_of(s * STRIPE, STRIPE)
            pltpu.sync_copy(wt_hbm.at[:, pl.ds(i0, STRIPE)],
                            src_v.at[:, pl.ds(0, STRIPE)])

            @pl.when(k > 0)
            def _():
                pltpu.make_async_copy(dst_v, wrm_hbm.at[pl.ds(0, STRIPE)],
                                      ssem).wait()

            def tok_body(tq, _):
                for ts in range(8):
                    i = tq * 8 + ts
                    iv = jnp.full((16,), i, jnp.int32)
                    r0 = plsc.load_gather(src_v, [iot, iv])
                    r1 = plsc.load_gather(src_v, [iot + 16, iv])
                    dst_v[i, pl.ds(0, 16)] = r0
                    dst_v[i, pl.ds(16, 16)] = r1
                return 0

            lax.fori_loop(0, STRIPE // 8, tok_body, 0)
            pltpu.async_copy(dst_v, wrm_hbm.at[pl.ds(i0, STRIPE)], ssem)
        return 0

    lax.fori_loop(0, STRIPES // NW + 1, stripe_body, 0)
    pltpu.make_async_copy(dst_v, wrm_hbm.at[pl.ds(0, STRIPE)], ssem).wait()


@functools.partial(
    pl.kernel,
    out_type=jax.ShapeDtypeStruct((SEQ, DT, NW, DR, BPW), jnp.float32),
    mesh=_MESH,
    compiler_params=pltpu.CompilerParams(use_tc_tiling_on_sc=False,
                                         needs_layout_passes=False),
    scratch_types=[
        pltpu.VMEM((SEQ, BPW), jnp.int32),
        pltpu.VMEM((SEQ, DIM), jnp.float32),
        [pltpu.VMEM((LPC * BPW, DIM), jnp.float32)] * 2,
        [pltpu.VMEM((LPC, DT, DR, BPW + 1), jnp.float32)] * 2,
        [pltpu.SemaphoreType.DMA] * 2,
        [pltpu.SemaphoreType.DMA] * 2,
    ],
)
def _sc_embed(w_hbm, xt_hbm, pos_hbm, out_hbm, idx_all, pos_v, rows, trans,
              gsem, ssem):
    wid = lax.axis_index("s") * NC + lax.axis_index("c")
    pltpu.sync_copy(pos_hbm, pos_v)
    b0 = pl.multiple_of(wid * BPW, BPW)
    pltpu.sync_copy(xt_hbm.at[:, pl.ds(b0, BPW)], idx_all)

    def fire_gather(c, rb):
        for j in range(LPC):
            pltpu.async_copy(w_hbm.at[idx_all.at[c * LPC + j]],
                             rows[rb].at[pl.ds(j * BPW, BPW)], gsem[rb])

    def drain_gather(rb):
        pltpu.make_async_copy(w_hbm.at[pl.ds(0, LPC * BPW)], rows[rb],
                              gsem[rb]).wait()

    def wait_store(tb):
        pltpu.make_async_copy(trans[tb].at[:, :, :, pl.ds(0, BPW)],
                              out_hbm.at[pl.ds(0, LPC), :, 0],
                              ssem[tb]).wait()

    iot = lax.iota(jnp.int32, 16)
    dtv = lax.shift_right_logical(iot, 3)
    dtv2 = dtv + 2
    drv = lax.bitwise_and(iot, 7)

    def compute(c, rb, tb):
        rbuf, tbuf = rows[rb], trans[tb]
        for lr in range(LPC):
            l = c * LPC + lr
            p0 = pos_v[l, pl.ds(0, 16)]
            p1 = pos_v[l, pl.ds(16, 16)]
            lv = jnp.full((16,), lr, jnp.int32)

            def tok_body(bq, _, lr=lr, p0=p0, p1=p1, lv=lv):
                for bs in range(8):
                    b = bq * 8 + bs
                    bv = jnp.full((16,), b, jnp.int32)
                    r0 = rbuf[lr * BPW + b, pl.ds(0, 16)] * SCALE + p0
                    r1 = rbuf[lr * BPW + b, pl.ds(16, 16)] * SCALE + p1
                    plsc.store_scatter(tbuf, [lv, dtv, drv, bv], r0)
                    plsc.store_scatter(tbuf, [lv, dtv2, drv, bv], r1)
                return 0

            lax.fori_loop(0, BPW // 8, tok_body, 0)

    fire_gather(0, 0)

    def pair_body(p, _):
        for rb in range(2):
            c = p * 2 + rb

            @pl.when(c + 1 < CHUNKS)
            def _(c=c, rb=rb):
                fire_gather(c + 1, 1 - rb)

            drain_gather(rb)

            @pl.when(c >= 2)
            def _(rb=rb):
                wait_store(rb)

            compute(c, rb, rb)
            for lr in range(LPC):
                pltpu.async_copy(
                    trans[rb].at[lr, :, :, pl.ds(0, BPW)],
                    out_hbm.at[c * LPC + lr, :, wid],
                    ssem[rb])
        return 0

    lax.fori_loop(0, CHUNKS // 2, pair_body, 0)
    wait_store(0)
    wait_store(1)


def kernel(x, W):
    pos = jnp.asarray(_make_pos_table())
    xt = x.T
    wrm = _sc_transpose(W.T)
    out5 = _sc_embed(wrm, xt, pos)
    return out5.transpose(2, 4, 0, 1, 3).reshape(BATCH, SEQ, DIM)


# speedup vs baseline: 3.4103x; 3.4103x over previous
import functools
import math

import jax
import jax.numpy as jnp
import numpy as np
from jax import lax
from jax.experimental import pallas as pl
from jax.experimental.pallas import tpu as pltpu
from jax.experimental.pallas import tpu_sc as plsc

VOCAB = 1000000
DIM = 32
MAX_LEN = 200
BATCH = 4096
SEQ = 200

NC, NS = 2, 16
NW = NC * NS
BPW = BATCH // NW
LPC = 4
CHUNKS = SEQ // LPC
DT, DR = DIM // 8, 8
SCALE = math.sqrt(DIM)


def _make_pos_table():
    para = np.arange(MAX_LEN).reshape(-1, 1) / np.power(
        10000.0, np.arange(0, DIM, 2) / DIM)
    pos = np.zeros((MAX_LEN, DIM), dtype=np.float32)
    pos[:, 0::2] = np.sin(para)
    pos[:, 1::2] = np.cos(para)
    return pos


_MESH = plsc.VectorSubcoreMesh(core_axis_name="c", subcore_axis_name="s",
                               num_cores=NC, num_subcores=NS)

TCHUNK = 768
TCHUNKS = 999936 // TCHUNK
TROWS = TCHUNK // 4
SRCPAD = 769


@functools.partial(
    pl.kernel,
    out_type=jax.ShapeDtypeStruct((250000, 128), jnp.float32),
    mesh=_MESH,
    compiler_params=pltpu.CompilerParams(use_tc_tiling_on_sc=True,
                                         needs_layout_passes=False),
    scratch_types=[
        [pltpu.VMEM((DIM, SRCPAD), jnp.float32)] * 2,
        [pltpu.VMEM((TROWS, 128), jnp.float32)] * 2,
        pltpu.VMEM((64, DIM), jnp.float32),
        [pltpu.SemaphoreType.DMA] * 2,
        [pltpu.SemaphoreType.DMA] * 2,
    ],
)
def _sc_transpose(wt_hbm, wtail_hbm, w4_hbm, src, stg, tail_v, gsem, ssem):
    wid = lax.axis_index("s") * NC + lax.axis_index("c")
    iot = lax.iota(jnp.int32, 16)

    @pl.when(wid == 0)
    def _():
        pltpu.sync_copy(wtail_hbm, tail_v)

        def tail_body(j, _):
            r = lax.shift_right_logical(j, 2)
            co = lax.bitwise_and(j, 3) * 32
            stg[0][r, pl.ds(co, 16)] = tail_v[j, pl.ds(0, 16)]
            stg[0][r, pl.ds(co + 16, 16)] = tail_v[j, pl.ds(16, 16)]
            return 0

        lax.fori_loop(0, 64, tail_body, 0)
        pltpu.sync_copy(stg[0].at[pl.ds(0, 16)],
                        w4_hbm.at[pl.ds(249984, 16)])

    def fire_in(s, b):
        i0 = pl.multiple_of(s * TCHUNK, TCHUNK)
        pltpu.async_copy(wt_hbm.at[:, pl.ds(i0, TCHUNK)],
                         src[b].at[:, pl.ds(0, TCHUNK)], gsem[b])

    def compute(b):
        sv, st = src[b], stg[b]

        def tok_body(tq, _):
            for ts in range(8):
                i = tq * 8 + ts
                iv = jnp.full((16,), i, jnp.int32)
                r0 = plsc.load_gather(sv, [iot, iv])
                r1 = plsc.load_gather(sv, [iot + 16, iv])
                r = lax.shift_right_logical(i, 2)
                co = lax.bitwise_and(i, 3) * 32
                st[r, pl.ds(co, 16)] = r0
                st[r, pl.ds(co + 16, 16)] = r1
            return 0

        lax.fori_loop(0, TCHUNK // 8, tok_body, 0)

    def store_out(s, b):
        r0 = pl.multiple_of(s * TROWS, TROWS)
        pltpu.async_copy(stg[b], w4_hbm.at[pl.ds(r0, TROWS)], ssem[b])

    def wait_out(b):
        pltpu.make_async_copy(w4_hbm.at[pl.ds(0, TROWS)],
                              w4_hbm.at[pl.ds(0, TROWS)], ssem[b]).wait()

    fire_in(wid, 0)

    def pair_body(p, _):
        for b in range(2):
            k = p * 2 + b
            s = k * NW + wid

            @pl.when(s + NW < TCHUNKS)
            def _(b=b, k=k, s=s):
                fire_in(s + NW, 1 - b)

            @pl.when(s < TCHUNKS)
            def _(b=b, k=k, s=s):
                pltpu.make_async_copy(wt_hbm.at[:, pl.ds(0, TCHUNK)],
                                      src[b].at[:, pl.ds(0, TCHUNK)],
                                      gsem[b]).wait()

                @pl.when(k >= 2)
                def _():
                    wait_out(b)

                compute(b)
                store_out(s, b)
        return 0

    lax.fori_loop(0, (TCHUNKS // NW + 2) // 2, pair_body, 0)
    wait_out(0)
    wait_out(1)


def _tail_rows(W):
    return lax.slice(W, (999936, 0), (VOCAB, DIM))


@functools.partial(
    pl.kernel,
    out_type=jax.ShapeDtypeStruct((SEQ, DT, NW, DR, BPW), jnp.float32),
    mesh=_MESH,
    compiler_params=pltpu.CompilerParams(use_tc_tiling_on_sc=False,
                                         needs_layout_passes=False),
    scratch_types=[
        pltpu.VMEM((SEQ, BPW), jnp.int32),
        pltpu.VMEM((SEQ, DIM), jnp.float32),
        [pltpu.VMEM((LPC * BPW, DIM), jnp.float32)] * 2,
        [pltpu.VMEM((LPC, DT, DR, BPW + 1), jnp.float32)] * 2,
        [pltpu.SemaphoreType.DMA] * 2,
        [pltpu.SemaphoreType.DMA] * 2,
    ],
)
def _sc_embed(w_hbm, xt_hbm, pos_hbm, out_hbm, idx_all, pos_v, rows, trans,
              gsem, ssem):
    wid = lax.axis_index("s") * NC + lax.axis_index("c")
    pltpu.sync_copy(pos_hbm, pos_v)
    b0 = pl.multiple_of(wid * BPW, BPW)
    pltpu.sync_copy(xt_hbm.at[:, pl.ds(b0, BPW)], idx_all)

    def fire_gather(c, rb):
        for j in range(LPC):
            pltpu.async_copy(w_hbm.at[idx_all.at[c * LPC + j]],
                             rows[rb].at[pl.ds(j * BPW, BPW)], gsem[rb])

    def drain_gather(rb):
        pltpu.make_async_copy(w_hbm.at[pl.ds(0, LPC * BPW)], rows[rb],
                              gsem[rb]).wait()

    def wait_store(tb):
        pltpu.make_async_copy(trans[tb].at[:, :, :, pl.ds(0, BPW)],
                              out_hbm.at[pl.ds(0, LPC), :, 0],
                              ssem[tb]).wait()

    iot = lax.iota(jnp.int32, 16)
    dtv = lax.shift_right_logical(iot, 3)
    dtv2 = dtv + 2
    drv = lax.bitwise_and(iot, 7)

    def compute(c, rb, tb):
        rbuf, tbuf = rows[rb], trans[tb]
        for lr in range(LPC):
            l = c * LPC + lr
            p0 = pos_v[l, pl.ds(0, 16)]
            p1 = pos_v[l, pl.ds(16, 16)]
            lv = jnp.full((16,), lr, jnp.int32)

            def tok_body(bq, _, lr=lr, p0=p0, p1=p1, lv=lv):
                for bs in range(8):
                    b = bq * 8 + bs
                    bv = jnp.full((16,), b, jnp.int32)
                    r0 = rbuf[lr * BPW + b, pl.ds(0, 16)] * SCALE + p0
                    r1 = rbuf[lr * BPW + b, pl.ds(16, 16)] * SCALE + p1
                    plsc.store_scatter(tbuf, [lv, dtv, drv, bv], r0)
                    plsc.store_scatter(tbuf, [lv, dtv2, drv, bv], r1)
                return 0

            lax.fori_loop(0, BPW // 8, tok_body, 0)

    fire_gather(0, 0)

    def pair_body(p, _):
        for rb in range(2):
            c = p * 2 + rb

            @pl.when(c + 1 < CHUNKS)
            def _(c=c, rb=rb):
                fire_gather(c + 1, 1 - rb)

            drain_gather(rb)

            @pl.when(c >= 2)
            def _(rb=rb):
                wait_store(rb)

            compute(c, rb, rb)
            for lr in range(LPC):
                pltpu.async_copy(
                    trans[rb].at[lr, :, :, pl.ds(0, BPW)],
                    out_hbm.at[c * LPC + lr, :, wid],
                    ssem[rb])
        return 0

    lax.fori_loop(0, CHUNKS // 2, pair_body, 0)
    wait_store(0)
    wait_store(1)


def kernel(x, W):
    pos = jnp.asarray(_make_pos_table())
    xt = x.T
    w4 = _sc_transpose(W.T, _tail_rows(W))
    out5 = _sc_embed(w4.reshape(VOCAB, DIM), xt, pos)
    return out5.transpose(2, 4, 0, 1, 3).reshape(BATCH, SEQ, DIM)


# speedup vs baseline: 3.4127x; 1.0007x over previous
import functools
import math

import jax
import jax.numpy as jnp
import numpy as np
from jax import lax
from jax.experimental import pallas as pl
from jax.experimental.pallas import tpu as pltpu
from jax.experimental.pallas import tpu_sc as plsc

VOCAB = 1000000
DIM = 32
MAX_LEN = 200
BATCH = 4096
SEQ = 200

NC, NS = 2, 16
NW = NC * NS
BPW = BATCH // NW
LPC = 4
CHUNKS = SEQ // LPC
DT, DR = DIM // 8, 8
SCALE = math.sqrt(DIM)


def _make_pos_table():
    para = np.arange(MAX_LEN).reshape(-1, 1) / np.power(
        10000.0, np.arange(0, DIM, 2) / DIM)
    pos = np.zeros((MAX_LEN, DIM), dtype=np.float32)
    pos[:, 0::2] = np.sin(para)
    pos[:, 1::2] = np.cos(para)
    return pos


_MESH = plsc.VectorSubcoreMesh(core_axis_name="c", subcore_axis_name="s",
                               num_cores=NC, num_subcores=NS)

TCHUNK = 768
TCHUNKS = 999936 // TCHUNK
TROWS = TCHUNK // 4
SRCPAD = 769


@functools.partial(
    pl.kernel,
    out_type=jax.ShapeDtypeStruct((250000, 128), jnp.float32),
    mesh=_MESH,
    compiler_params=pltpu.CompilerParams(use_tc_tiling_on_sc=True,
                                         needs_layout_passes=False),
    scratch_types=[
        [pltpu.VMEM((DIM, SRCPAD), jnp.float32)] * 2,
        [pltpu.VMEM((TROWS, 128), jnp.float32)] * 2,
        pltpu.VMEM((64, DIM), jnp.float32),
        [pltpu.SemaphoreType.DMA] * 2,
        [pltpu.SemaphoreType.DMA] * 2,
    ],
)
def _sc_transpose(wt_hbm, wtail_hbm, w4_hbm, src, stg, tail_v, gsem, ssem):
    wid = lax.axis_index("s") * NC + lax.axis_index("c")
    iot = lax.iota(jnp.int32, 16)

    @pl.when(wid == 0)
    def _():
        pltpu.sync_copy(wtail_hbm, tail_v)

        def tail_body(j, _):
            r = lax.shift_right_logical(j, 2)
            co = lax.bitwise_and(j, 3) * 32
            stg[0][r, pl.ds(co, 16)] = tail_v[j, pl.ds(0, 16)]
            stg[0][r, pl.ds(co + 16, 16)] = tail_v[j, pl.ds(16, 16)]
            return 0

        lax.fori_loop(0, 64, tail_body, 0)
        pltpu.sync_copy(stg[0].at[pl.ds(0, 16)],
                        w4_hbm.at[pl.ds(249984, 16)])

    def fire_in(s, b):
        i0 = pl.multiple_of(s * TCHUNK, TCHUNK)
        pltpu.async_copy(wt_hbm.at[:, pl.ds(i0, TCHUNK)],
                         src[b].at[:, pl.ds(0, TCHUNK)], gsem[b])

    def compute(b):
        sv, st = src[b], stg[b]

        def tok_body(tq, _):
            for ts in range(8):
                i = tq * 8 + ts
                iv = jnp.full((16,), i, jnp.int32)
                r0 = plsc.load_gather(sv, [iot, iv])
                r1 = plsc.load_gather(sv, [iot + 16, iv])
                r = tq * 2 + ts // 4
                co = (ts % 4) * 32
                st[r, pl.ds(co, 16)] = r0
                st[r, pl.ds(co + 16, 16)] = r1
            return 0

        lax.fori_loop(0, TCHUNK // 8, tok_body, 0)

    def store_out(s, b):
        r0 = pl.multiple_of(s * TROWS, TROWS)
        pltpu.async_copy(stg[b], w4_hbm.at[pl.ds(r0, TROWS)], ssem[b])

    def wait_out(b):
        pltpu.make_async_copy(w4_hbm.at[pl.ds(0, TROWS)],
                              w4_hbm.at[pl.ds(0, TROWS)], ssem[b]).wait()

    fire_in(wid, 0)

    def pair_body(p, _):
        for b in range(2):
            k = p * 2 + b
            s = k * NW + wid

            @pl.when(s + NW < TCHUNKS)
            def _(b=b, k=k, s=s):
                fire_in(s + NW, 1 - b)

            @pl.when(s < TCHUNKS)
            def _(b=b, k=k, s=s):
                pltpu.make_async_copy(wt_hbm.at[:, pl.ds(0, TCHUNK)],
                                      src[b].at[:, pl.ds(0, TCHUNK)],
                                      gsem[b]).wait()

                @pl.when(k >= 2)
                def _():
                    wait_out(b)

                compute(b)
                store_out(s, b)
        return 0

    lax.fori_loop(0, (TCHUNKS // NW + 2) // 2, pair_body, 0)
    wait_out(0)
    wait_out(1)


def _tail_rows(W):
    return lax.slice(W, (999936, 0), (VOCAB, DIM))


@functools.partial(
    pl.kernel,
    out_type=jax.ShapeDtypeStruct((SEQ, DT, NW, DR, BPW), jnp.float32),
    mesh=_MESH,
    compiler_params=pltpu.CompilerParams(use_tc_tiling_on_sc=False,
                                         needs_layout_passes=False),
    scratch_types=[
        pltpu.VMEM((SEQ, BPW), jnp.int32),
        pltpu.VMEM((SEQ, DIM), jnp.float32),
        [pltpu.VMEM((LPC * BPW, DIM), jnp.float32)] * 2,
        [pltpu.VMEM((LPC, DT, DR, BPW + 1), jnp.float32)] * 2,
        [pltpu.SemaphoreType.DMA] * 2,
        [pltpu.SemaphoreType.DMA] * 2,
    ],
)
def _sc_embed(w_hbm, xt_hbm, pos_hbm, out_hbm, idx_all, pos_v, rows, trans,
              gsem, ssem):
    wid = lax.axis_index("s") * NC + lax.axis_index("c")
    pltpu.sync_copy(pos_hbm, pos_v)
    b0 = pl.multiple_of(wid * BPW, BPW)
    pltpu.sync_copy(xt_hbm.at[:, pl.ds(b0, BPW)], idx_all)

    def fire_gather(c, rb):
        for j in range(LPC):
            pltpu.async_copy(w_hbm.at[idx_all.at[c * LPC + j]],
                             rows[rb].at[pl.ds(j * BPW, BPW)], gsem[rb])

    def drain_gather(rb):
        pltpu.make_async_copy(w_hbm.at[pl.ds(0, LPC * BPW)], rows[rb],
                              gsem[rb]).wait()

    def wait_store(tb):
        pltpu.make_async_copy(trans[tb].at[:, :, :, pl.ds(0, BPW)],
                              out_hbm.at[pl.ds(0, LPC), :, 0],
                              ssem[tb]).wait()

    iot = lax.iota(jnp.int32, 16)
    dtv = lax.shift_right_logical(iot, 3)
    dtv2 = dtv + 2
    drv = lax.bitwise_and(iot, 7)

    def compute(c, rb, tb):
        rbuf, tbuf = rows[rb], trans[tb]
        for lr in range(LPC):
            l = c * LPC + lr
            p0 = pos_v[l, pl.ds(0, 16)]
            p1 = pos_v[l, pl.ds(16, 16)]
            lv = jnp.full((16,), lr, jnp.int32)

            def tok_body(bq, _, lr=lr, p0=p0, p1=p1, lv=lv):
                for bs in range(8):
                    b = bq * 8 + bs
                    bv = jnp.full((16,), b, jnp.int32)
                    r0 = rbuf[lr * BPW + b, pl.ds(0, 16)] * SCALE + p0
                    r1 = rbuf[lr * BPW + b, pl.ds(16, 16)] * SCALE + p1
                    plsc.store_scatter(tbuf, [lv, dtv, drv, bv], r0)
                    plsc.store_scatter(tbuf, [lv, dtv2, drv, bv], r1)
                return 0

            lax.fori_loop(0, BPW // 8, tok_body, 0)

    fire_gather(0, 0)

    def pair_body(p, _):
        for rb in range(2):
            c = p * 2 + rb

            @pl.when(c + 1 < CHUNKS)
            def _(c=c, rb=rb):
                fire_gather(c + 1, 1 - rb)

            drain_gather(rb)

            @pl.when(c >= 2)
            def _(rb=rb):
                wait_store(rb)

            compute(c, rb, rb)
            for lr in range(LPC):
                pltpu.async_copy(
                    trans[rb].at[lr, :, :, pl.ds(0, BPW)],
                    out_hbm.at[c * LPC + lr, :, wid],
                    ssem[rb])
        return 0

    lax.fori_loop(0, CHUNKS // 2, pair_body, 0)
    wait_store(0)
    wait_store(1)


def kernel(x, W):
    pos = jnp.asarray(_make_pos_table())
    xt = x.T
    w4 = _sc_transpose(W.T, _tail_rows(W))
    out5 = _sc_embed(w4.reshape(VOCAB, DIM), xt, pos)
    return out5.transpose(2, 4, 0, 1, 3).reshape(BATCH, SEQ, DIM)
